# Initial kernel scaffold; baseline (speedup 1.0000x reference)
import jax
import jax.numpy as jnp
from jax.experimental import pallas as pl


def kernel(node_features, edge_index, edge_features, mW1, mb1, mW2, mb2, uW1, ub1, uW2, ub2):
    raise NotImplementedError("write your pallas kernel here")


# speedup vs baseline: 2.9307x; 2.9307x over previous
import functools

import jax
import jax.numpy as jnp
from jax import lax
from jax.experimental import pallas as pl
from jax.experimental.pallas import tpu as pltpu
from jax.experimental.pallas import tpu_sc as plsc

N_NODES = 10000
N_EDGES = 320000
NODE_DIM = 128
EDGE_DIM = 16
HIDDEN_DIM = 128

NP = 10240
NC = 2
NS = 16
NW = NC * NS
EPW = N_EDGES // NW
CH = 80
NCH = EPW // CH
RPT = NP // NS
@functools.lru_cache(maxsize=None)
def _sc_mesh():
    return plsc.VectorSubcoreMesh(
        core_axis_name="c", subcore_axis_name="s", num_cores=NC, num_subcores=NS
    )


def _gather_body(nf_hbm, idx_hbm, out_hbm, idx_v, rows_v, sem):
    c = lax.axis_index("c")
    s = lax.axis_index("s")
    wid = c * NS + s
    pltpu.sync_copy(idx_hbm.at[wid], idx_v)

    def body(j, carry):
        pltpu.async_copy(nf_hbm.at[idx_v.at[j]], rows_v, sem).wait()
        pltpu.sync_copy(rows_v, out_hbm.at[pl.ds(wid * EPW + j * CH, CH)])
        return carry

    lax.fori_loop(0, NCH, body, 0, unroll=False)


@functools.lru_cache(maxsize=None)
def _gather():
    return pl.kernel(
        _gather_body,
        out_type=jax.ShapeDtypeStruct((N_EDGES, NODE_DIM), jnp.float32),
        mesh=_sc_mesh(),
        scratch_types=[
            pltpu.VMEM((NCH, CH), jnp.int32),
            pltpu.VMEM((CH, NODE_DIM), jnp.float32),
            pltpu.SemaphoreType.DMA,
        ],
    )


def _scatter_body(msg_hbm, idx_hbm, out_hbm, idx_v, msg_v, acc_sh, sem):
    c = lax.axis_index("c")
    s = lax.axis_index("s")
    wid = c * NS + s

    def zrow(i, carry):
        def zcol(k, carry2):
            msg_v[i, pl.ds(k * 16, 16)] = jnp.zeros((16,), jnp.float32)
            return carry2
        return lax.fori_loop(0, NODE_DIM // 16, zcol, carry, unroll=False)

    lax.fori_loop(0, CH, zrow, 0, unroll=False)

    def zslab(t, carry):
        pltpu.sync_copy(msg_v, acc_sh.at[pl.ds(s * RPT + t * CH, CH)])
        return carry

    lax.fori_loop(0, RPT // CH, zslab, 0, unroll=False)
    plsc.subcore_barrier()

    pltpu.sync_copy(idx_hbm.at[wid], idx_v)

    def body(j, carry):
        pltpu.async_copy(msg_hbm.at[pl.ds(wid * EPW + j * CH, CH)], msg_v, sem).wait()
        pltpu.sync_copy(msg_v, acc_sh.at[idx_v.at[j]], add=True)
        return carry

    lax.fori_loop(0, NCH, body, 0, unroll=False)
    plsc.subcore_barrier()

    def rb(t, carry):
        pltpu.sync_copy(acc_sh.at[pl.ds(s * RPT + t * CH, CH)], msg_v)
        pltpu.sync_copy(msg_v, out_hbm.at[c, pl.ds(s * RPT + t * CH, CH)])
        return carry

    lax.fori_loop(0, RPT // CH, rb, 0, unroll=False)


@functools.lru_cache(maxsize=None)
def _scatter():
    return pl.kernel(
        _scatter_body,
        out_type=jax.ShapeDtypeStruct((NC, NP, NODE_DIM), jnp.float32),
        mesh=_sc_mesh(),
        scratch_types=[
            pltpu.VMEM((NCH, CH), jnp.int32),
            pltpu.VMEM((CH, NODE_DIM), jnp.float32),
            pltpu.VMEM_SHARED((NP, NODE_DIM), jnp.float32),
            pltpu.SemaphoreType.DMA,
        ],
    )


BE = 2000


def _edge_mlp_body(g_ref, e_ref, w1a_ref, w1b_ref, b1_ref, w2_ref, b2_ref, o_ref):
    h = jnp.dot(g_ref[...], w1a_ref[...], preferred_element_type=jnp.float32)
    h = h + jnp.dot(e_ref[...], w1b_ref[...], preferred_element_type=jnp.float32)
    h = jnp.maximum(h + b1_ref[...], 0.0)
    o_ref[...] = jnp.dot(h, w2_ref[...], preferred_element_type=jnp.float32) + b2_ref[...]


def _edge_mlp(gathered, edge_features, w1a, w1b, b1, w2, b2):
    grid = (N_EDGES // BE,)
    full = lambda shape: pl.BlockSpec(shape, lambda i: (0, 0))
    return pl.pallas_call(
        _edge_mlp_body,
        grid=grid,
        in_specs=[
            pl.BlockSpec((BE, NODE_DIM), lambda i: (i, 0)),
            pl.BlockSpec((BE, EDGE_DIM), lambda i: (i, 0)),
            full((NODE_DIM, HIDDEN_DIM)),
            full((EDGE_DIM, HIDDEN_DIM)),
            full((1, HIDDEN_DIM)),
            full((HIDDEN_DIM, HIDDEN_DIM)),
            full((1, HIDDEN_DIM)),
        ],
        out_specs=pl.BlockSpec((BE, HIDDEN_DIM), lambda i: (i, 0)),
        out_shape=jax.ShapeDtypeStruct((N_EDGES, HIDDEN_DIM), jnp.float32),
        compiler_params=pltpu.CompilerParams(
            dimension_semantics=("arbitrary",),
        ),
    )(gathered, edge_features, w1a, w1b, b1, w2, b2)


BN = 1280


def _update_body(nf_ref, p_ref, w1a_ref, w1b_ref, b1_ref, w2_ref, b2_ref, o_ref):
    agg = p_ref[0] + p_ref[1]
    h = jnp.dot(nf_ref[...], w1a_ref[...], preferred_element_type=jnp.float32)
    h = h + jnp.dot(agg, w1b_ref[...], preferred_element_type=jnp.float32)
    h = jnp.maximum(h + b1_ref[...], 0.0)
    o_ref[...] = jnp.dot(h, w2_ref[...], preferred_element_type=jnp.float32) + b2_ref[...]


def _update_mlp(nf_pad, partials, w1a, w1b, b1, w2, b2):
    grid = (NP // BN,)
    full = lambda shape: pl.BlockSpec(shape, lambda i: tuple(0 for _ in shape))
    return pl.pallas_call(
        _update_body,
        grid=grid,
        in_specs=[
            pl.BlockSpec((BN, NODE_DIM), lambda i: (i, 0)),
            pl.BlockSpec((NC, BN, NODE_DIM), lambda i: (0, i, 0)),
            full((NODE_DIM, HIDDEN_DIM)),
            full((HIDDEN_DIM, HIDDEN_DIM)),
            full((1, HIDDEN_DIM)),
            full((HIDDEN_DIM, NODE_DIM)),
            full((1, NODE_DIM)),
        ],
        out_specs=pl.BlockSpec((BN, NODE_DIM), lambda i: (i, 0)),
        out_shape=jax.ShapeDtypeStruct((NP, NODE_DIM), jnp.float32),
        compiler_params=pltpu.CompilerParams(
            dimension_semantics=("arbitrary",),
        ),
    )(nf_pad, partials, w1a, w1b, b1, w2, b2)


@jax.jit
def kernel(node_features, edge_index, edge_features, mW1, mb1, mW2, mb2, uW1, ub1, uW2, ub2):
    src = edge_index[0].astype(jnp.int32).reshape(NW, NCH, CH)
    dst = edge_index[1].astype(jnp.int32).reshape(NW, NCH, CH)
    nf_pad = jnp.pad(node_features, ((0, NP - N_NODES), (0, 0)))

    gathered = _gather()(nf_pad, src)
    messages = _edge_mlp(
        gathered, edge_features,
        mW1[:NODE_DIM], mW1[NODE_DIM:],
        mb1.reshape(1, HIDDEN_DIM), mW2, mb2.reshape(1, HIDDEN_DIM),
    )
    partials = _scatter()(messages, dst)
    out = _update_mlp(
        nf_pad, partials,
        uW1[:NODE_DIM], uW1[NODE_DIM:],
        ub1.reshape(1, HIDDEN_DIM), uW2, ub2.reshape(1, NODE_DIM),
    )
    return out[:N_NODES]


# speedup vs baseline: 3.6521x; 1.2461x over previous
import functools

import jax
import jax.numpy as jnp
from jax import lax
from jax.experimental import pallas as pl
from jax.experimental.pallas import tpu as pltpu
from jax.experimental.pallas import tpu_sc as plsc

N_NODES = 10000
N_EDGES = 320000
NODE_DIM = 128
EDGE_DIM = 16
HIDDEN_DIM = 128

NP = 10240
NC = 2
NS = 16
NW = NC * NS
EPW = N_EDGES // NW
CH = 80
NCH = EPW // CH
RPT = NP // NS
@functools.lru_cache(maxsize=None)
def _sc_mesh():
    return plsc.VectorSubcoreMesh(
        core_axis_name="c", subcore_axis_name="s", num_cores=NC, num_subcores=NS
    )


def _gather_body(nf_hbm, idx_hbm, out_hbm, idx_v, rows_a, rows_b, sem_a, sem_b):
    c = lax.axis_index("c")
    s = lax.axis_index("s")
    wid = c * NS + s
    base = wid * EPW
    pltpu.sync_copy(idx_hbm.at[wid], idx_v)

    def wait(buf, sem):
        pltpu.make_async_copy(nf_hbm.at[pl.ds(0, CH)], buf, sem).wait()

    pltpu.async_copy(nf_hbm.at[idx_v.at[0]], rows_a, sem_a)

    @pl.loop(0, NCH - 1, step=2)
    def _(j):
        pltpu.async_copy(nf_hbm.at[idx_v.at[j + 1]], rows_b, sem_b)
        wait(rows_a, sem_a)
        pltpu.sync_copy(rows_a, out_hbm.at[pl.ds(base + j * CH, CH)])
        pltpu.async_copy(nf_hbm.at[idx_v.at[j + 2]], rows_a, sem_a)
        wait(rows_b, sem_b)
        pltpu.sync_copy(rows_b, out_hbm.at[pl.ds(base + (j + 1) * CH, CH)])

    wait(rows_a, sem_a)
    pltpu.sync_copy(rows_a, out_hbm.at[pl.ds(base + (NCH - 1) * CH, CH)])


@functools.lru_cache(maxsize=None)
def _gather():
    return pl.kernel(
        _gather_body,
        out_type=jax.ShapeDtypeStruct((N_EDGES, NODE_DIM), jnp.float32),
        mesh=_sc_mesh(),
        scratch_types=[
            pltpu.VMEM((NCH, CH), jnp.int32),
            pltpu.VMEM((CH, NODE_DIM), jnp.float32),
            pltpu.VMEM((CH, NODE_DIM), jnp.float32),
            pltpu.SemaphoreType.DMA,
            pltpu.SemaphoreType.DMA,
        ],
    )


def _scatter_body(msg_hbm, idx_hbm, out_hbm, idx_v, msg_a, msg_b, acc_sh, sem_a, sem_b):
    c = lax.axis_index("c")
    s = lax.axis_index("s")
    wid = c * NS + s
    base = wid * EPW

    def zrow(i, carry):
        def zcol(k, carry2):
            msg_a[i, pl.ds(k * 16, 16)] = jnp.zeros((16,), jnp.float32)
            return carry2
        return lax.fori_loop(0, NODE_DIM // 16, zcol, carry, unroll=False)

    lax.fori_loop(0, CH, zrow, 0, unroll=False)

    def zslab(t, carry):
        pltpu.sync_copy(msg_a, acc_sh.at[pl.ds(s * RPT + t * CH, CH)])
        return carry

    lax.fori_loop(0, RPT // CH, zslab, 0, unroll=False)
    plsc.subcore_barrier()

    pltpu.sync_copy(idx_hbm.at[wid], idx_v)

    def wait(buf, sem):
        pltpu.make_async_copy(msg_hbm.at[pl.ds(0, CH)], buf, sem).wait()

    def load(j, buf, sem):
        pltpu.async_copy(msg_hbm.at[pl.ds(base + j * CH, CH)], buf, sem)

    load(0, msg_a, sem_a)

    @pl.loop(0, NCH - 1, step=2)
    def _(j):
        load(j + 1, msg_b, sem_b)
        wait(msg_a, sem_a)
        pltpu.sync_copy(msg_a, acc_sh.at[idx_v.at[j]], add=True)
        load(j + 2, msg_a, sem_a)
        wait(msg_b, sem_b)
        pltpu.sync_copy(msg_b, acc_sh.at[idx_v.at[j + 1]], add=True)

    wait(msg_a, sem_a)
    pltpu.sync_copy(msg_a, acc_sh.at[idx_v.at[NCH - 1]], add=True)
    plsc.subcore_barrier()

    def rb(t, carry):
        pltpu.sync_copy(acc_sh.at[pl.ds(s * RPT + t * CH, CH)], msg_a)
        pltpu.sync_copy(msg_a, out_hbm.at[c, pl.ds(s * RPT + t * CH, CH)])
        return carry

    lax.fori_loop(0, RPT // CH, rb, 0, unroll=False)


@functools.lru_cache(maxsize=None)
def _scatter():
    return pl.kernel(
        _scatter_body,
        out_type=jax.ShapeDtypeStruct((NC, NP, NODE_DIM), jnp.float32),
        mesh=_sc_mesh(),
        scratch_types=[
            pltpu.VMEM((NCH, CH), jnp.int32),
            pltpu.VMEM((CH, NODE_DIM), jnp.float32),
            pltpu.VMEM((CH, NODE_DIM), jnp.float32),
            pltpu.VMEM_SHARED((NP, NODE_DIM), jnp.float32),
            pltpu.SemaphoreType.DMA,
            pltpu.SemaphoreType.DMA,
        ],
    )


BE = 2000


def _edge_mlp_body(g_ref, e_ref, w1a_ref, w1b_ref, b1_ref, w2_ref, b2_ref, o_ref):
    h = jnp.dot(g_ref[...], w1a_ref[...], preferred_element_type=jnp.float32)
    h = h + jnp.dot(e_ref[...], w1b_ref[...], preferred_element_type=jnp.float32)
    h = jnp.maximum(h + b1_ref[...], 0.0)
    o_ref[...] = jnp.dot(h, w2_ref[...], preferred_element_type=jnp.float32) + b2_ref[...]


def _edge_mlp(gathered, edge_features, w1a, w1b, b1, w2, b2):
    grid = (N_EDGES // BE,)
    full = lambda shape: pl.BlockSpec(shape, lambda i: (0, 0))
    return pl.pallas_call(
        _edge_mlp_body,
        grid=grid,
        in_specs=[
            pl.BlockSpec((BE, NODE_DIM), lambda i: (i, 0)),
            pl.BlockSpec((BE, EDGE_DIM), lambda i: (i, 0)),
            full((NODE_DIM, HIDDEN_DIM)),
            full((EDGE_DIM, HIDDEN_DIM)),
            full((1, HIDDEN_DIM)),
            full((HIDDEN_DIM, HIDDEN_DIM)),
            full((1, HIDDEN_DIM)),
        ],
        out_specs=pl.BlockSpec((BE, HIDDEN_DIM), lambda i: (i, 0)),
        out_shape=jax.ShapeDtypeStruct((N_EDGES, HIDDEN_DIM), jnp.float32),
        compiler_params=pltpu.CompilerParams(
            dimension_semantics=("arbitrary",),
        ),
    )(gathered, edge_features, w1a, w1b, b1, w2, b2)


BN = 1280


def _update_body(nf_ref, p_ref, w1a_ref, w1b_ref, b1_ref, w2_ref, b2_ref, o_ref):
    agg = p_ref[0] + p_ref[1]
    h = jnp.dot(nf_ref[...], w1a_ref[...], preferred_element_type=jnp.float32)
    h = h + jnp.dot(agg, w1b_ref[...], preferred_element_type=jnp.float32)
    h = jnp.maximum(h + b1_ref[...], 0.0)
    o_ref[...] = jnp.dot(h, w2_ref[...], preferred_element_type=jnp.float32) + b2_ref[...]


def _update_mlp(nf_pad, partials, w1a, w1b, b1, w2, b2):
    grid = (NP // BN,)
    full = lambda shape: pl.BlockSpec(shape, lambda i: tuple(0 for _ in shape))
    return pl.pallas_call(
        _update_body,
        grid=grid,
        in_specs=[
            pl.BlockSpec((BN, NODE_DIM), lambda i: (i, 0)),
            pl.BlockSpec((NC, BN, NODE_DIM), lambda i: (0, i, 0)),
            full((NODE_DIM, HIDDEN_DIM)),
            full((HIDDEN_DIM, HIDDEN_DIM)),
            full((1, HIDDEN_DIM)),
            full((HIDDEN_DIM, NODE_DIM)),
            full((1, NODE_DIM)),
        ],
        out_specs=pl.BlockSpec((BN, NODE_DIM), lambda i: (i, 0)),
        out_shape=jax.ShapeDtypeStruct((NP, NODE_DIM), jnp.float32),
        compiler_params=pltpu.CompilerParams(
            dimension_semantics=("arbitrary",),
        ),
    )(nf_pad, partials, w1a, w1b, b1, w2, b2)


@jax.jit
def kernel(node_features, edge_index, edge_features, mW1, mb1, mW2, mb2, uW1, ub1, uW2, ub2):
    src = edge_index[0].astype(jnp.int32).reshape(NW, NCH, CH)
    dst = edge_index[1].astype(jnp.int32).reshape(NW, NCH, CH)
    nf_pad = jnp.pad(node_features, ((0, NP - N_NODES), (0, 0)))

    gathered = _gather()(nf_pad, src)
    messages = _edge_mlp(
        gathered, edge_features,
        mW1[:NODE_DIM], mW1[NODE_DIM:],
        mb1.reshape(1, HIDDEN_DIM), mW2, mb2.reshape(1, HIDDEN_DIM),
    )
    partials = _scatter()(messages, dst)
    out = _update_mlp(
        nf_pad, partials,
        uW1[:NODE_DIM], uW1[NODE_DIM:],
        ub1.reshape(1, HIDDEN_DIM), uW2, ub2.reshape(1, NODE_DIM),
    )
    return out[:N_NODES]
